# Initial kernel scaffold; baseline (speedup 1.0000x reference)
import jax
import jax.numpy as jnp
from jax.experimental import pallas as pl


def kernel(word_ids, position_ids, adj, edge_types, cls_node, tok_table, pos_table, edge_table, Wq, Wk, Wv, Wo, ln1_g, ln1_b, W1, b1, W2, b2, ln2_g, ln2_b, Wc, bc):
    raise NotImplementedError("write your pallas kernel here")


# speedup vs baseline: 2238.6257x; 2238.6257x over previous
import functools

import jax
import jax.numpy as jnp
from jax import lax
from jax.experimental import pallas as pl
from jax.experimental.pallas import tpu as pltpu
from jax.experimental.pallas import tpu_sc as plsc

N = 2048
D = 128
H = 8
DH = 16
FF = 512
NET = 17
NCLASS = 5
B = 16
TN = 256
SCALE = 1.0 / (DH ** 0.5)


def _embed_gather(word_ids, position_ids, tok_table, pos_table):
  info = plsc.get_sparse_core_info()
  nw = info.num_cores * info.num_subcores
  rpw = N // nw

  mesh = plsc.VectorSubcoreMesh(core_axis_name="c", subcore_axis_name="s")

  @functools.partial(
      pl.kernel,
      mesh=mesh,
      out_type=[
          jax.ShapeDtypeStruct((N, D), jnp.float32),
          jax.ShapeDtypeStruct((N, D), jnp.float32),
      ],
      scratch_types=[
          pltpu.VMEM((rpw,), jnp.int32),
          pltpu.VMEM((rpw,), jnp.int32),
          pltpu.VMEM((rpw, D), jnp.float32),
          pltpu.VMEM((rpw, D), jnp.float32),
          pltpu.SemaphoreType.DMA,
          pltpu.SemaphoreType.DMA,
      ],
  )
  def k(wid_hbm, pid_hbm, tok_hbm, pos_hbm, tout_hbm, pout_hbm,
        widx, pidx, trows, prows, sem1, sem2):
    w = lax.axis_index("s") * info.num_cores + lax.axis_index("c")
    base = w * rpw
    pltpu.sync_copy(wid_hbm.at[pl.ds(base, rpw)], widx)
    pltpu.sync_copy(pid_hbm.at[pl.ds(base, rpw)], pidx)
    c1 = pltpu.async_copy(tok_hbm.at[widx], trows, sem1)
    c2 = pltpu.async_copy(pos_hbm.at[pidx], prows, sem2)
    c1.wait()
    c2.wait()
    pltpu.sync_copy(trows, tout_hbm.at[pl.ds(base, rpw)])
    pltpu.sync_copy(prows, pout_hbm.at[pl.ds(base, rpw)])

  return k(word_ids, position_ids, tok_table, pos_table)


def _qkv0_body(t_ref, p_ref, wq_ref, wk_ref, wv_ref,
               h_ref, q_ref, k_ref, v_ref):
  hh = t_ref[...] + p_ref[...]
  h_ref[...] = hh
  q_ref[...] = jnp.dot(hh, wq_ref[...], preferred_element_type=jnp.float32)
  k_ref[...] = jnp.dot(hh, wk_ref[...], preferred_element_type=jnp.float32)
  v_ref[...] = jnp.dot(hh, wv_ref[...], preferred_element_type=jnp.float32)


def _qkv1_body(h_ref, wq_ref, wk_ref, wv_ref, q_ref, k_ref, v_ref):
  hh = h_ref[...]
  q_ref[...] = jnp.dot(hh, wq_ref[...], preferred_element_type=jnp.float32)
  k_ref[...] = jnp.dot(hh, wk_ref[...], preferred_element_type=jnp.float32)
  v_ref[...] = jnp.dot(hh, wv_ref[...], preferred_element_type=jnp.float32)


def _row_spec():
  return pl.BlockSpec((TN, D), lambda i: (i, 0))


def _full_spec(shape):
  return pl.BlockSpec(shape, lambda i: tuple(0 for _ in shape))


def _qkv0(t, p, wq, wk, wv):
  return pl.pallas_call(
      _qkv0_body,
      grid=(N // TN,),
      in_specs=[_row_spec(), _row_spec(),
                _full_spec((D, D)), _full_spec((D, D)), _full_spec((D, D))],
      out_specs=[_row_spec(), _row_spec(), _row_spec(), _row_spec()],
      out_shape=[jax.ShapeDtypeStruct((N, D), jnp.float32)] * 4,
  )(t, p, wq, wk, wv)


def _qkv1(h, wq, wk, wv):
  return pl.pallas_call(
      _qkv1_body,
      grid=(N // TN,),
      in_specs=[_row_spec(),
                _full_spec((D, D)), _full_spec((D, D)), _full_spec((D, D))],
      out_specs=[_row_spec(), _row_spec(), _row_spec()],
      out_shape=[jax.ShapeDtypeStruct((N, D), jnp.float32)] * 3,
  )(h, wq, wk, wv)


def _layer_norm(x, g, b, eps=1e-5):
  m = jnp.mean(x, axis=-1, keepdims=True)
  v = jnp.mean((x - m) * (x - m), axis=-1, keepdims=True)
  return (x - m) / jnp.sqrt(v + eps) * g + b


def _attn_body(q_ref, k_ref, v_ref, h_ref, adj_ref, et_ref, ek_ref,
               wo_ref, ln1g_ref, ln1b_ref, w1_ref, b1_ref, w2_ref, b2_ref,
               ln2g_ref, ln2b_ref, out_ref):
  q = q_ref[...]
  hin = h_ref[...]
  mask = adj_ref[...] > 0.0
  et = et_ref[...]
  neg = jnp.float32(-1e9)

  ctx_parts = []
  for hh in range(H):
    sl = slice(hh * DH, (hh + 1) * DH)
    qh = q[:, sl]
    kh = k_ref[:, sl]
    vh = v_ref[:, sl]
    ekh = ek_ref[:, sl]
    s = lax.dot_general(qh, kh, (((1,), (1,)), ((), ())),
                        preferred_element_type=jnp.float32) * SCALE
    qe = lax.dot_general(qh, ekh, (((1,), (1,)), ((), ())),
                         preferred_element_type=jnp.float32) * SCALE
    bias = jnp.take_along_axis(qe, et, axis=1, mode="promise_in_bounds")
    s = jnp.where(mask, s + bias, neg)
    m = jnp.max(s, axis=1, keepdims=True)
    e = jnp.exp(s - m)
    denom = jnp.sum(e, axis=1, keepdims=True)
    attn = e / denom
    ctx_parts.append(
        lax.dot_general(attn, vh, (((1,), (0,)), ((), ())),
                        preferred_element_type=jnp.float32))
  ctx = jnp.concatenate(ctx_parts, axis=1)

  x = hin + jnp.dot(ctx, wo_ref[...], preferred_element_type=jnp.float32)
  x = _layer_norm(x, ln1g_ref[...], ln1b_ref[...])
  ffn = jnp.dot(
      jnp.maximum(
          jnp.dot(x, w1_ref[...], preferred_element_type=jnp.float32)
          + b1_ref[...],
          0.0),
      w2_ref[...], preferred_element_type=jnp.float32) + b2_ref[...]
  out_ref[...] = _layer_norm(x + ffn, ln2g_ref[...], ln2b_ref[...])


def _attn_layer(q, k, v, h, adj, et, ek, wo, ln1g, ln1b, w1, b1, w2, b2,
                ln2g, ln2b):
  return pl.pallas_call(
      _attn_body,
      grid=(N // TN,),
      in_specs=[
          _row_spec(),
          _full_spec((N, D)),
          _full_spec((N, D)),
          _row_spec(),
          pl.BlockSpec((TN, N), lambda i: (i, 0)),
          pl.BlockSpec((TN, N), lambda i: (i, 0)),
          _full_spec((NET, D)),
          _full_spec((D, D)),
          _full_spec((1, D)), _full_spec((1, D)),
          _full_spec((D, FF)), _full_spec((1, FF)),
          _full_spec((FF, D)), _full_spec((1, D)),
          _full_spec((1, D)), _full_spec((1, D)),
      ],
      out_specs=_row_spec(),
      out_shape=jax.ShapeDtypeStruct((N, D), jnp.float32),
  )(q, k, v, h, adj, et, ek, wo, ln1g, ln1b, w1, b1, w2, b2, ln2g, ln2b)


def _cls_body(h_ref, cls_ref, wc_ref, bc_ref, out_ref):
  ids = lax.broadcasted_iota(jnp.int32, (B, N), 1)
  oh = (ids == cls_ref[...]).astype(jnp.float32)
  ch = jnp.dot(oh, h_ref[...], preferred_element_type=jnp.float32)
  out_ref[...] = (
      jnp.dot(ch, wc_ref[...], preferred_element_type=jnp.float32)
      + bc_ref[...])


def _cls_head(h, cls_node, wc_pad, bc_pad):
  return pl.pallas_call(
      _cls_body,
      out_shape=jax.ShapeDtypeStruct((B, D), jnp.float32),
  )(h, cls_node, wc_pad, bc_pad)


def kernel(word_ids, position_ids, adj, edge_types, cls_node, tok_table,
           pos_table, edge_table, Wq, Wk, Wv, Wo, ln1_g, ln1_b, W1, b1, W2,
           b2, ln2_g, ln2_b, Wc, bc):
  word_ids = word_ids.astype(jnp.int32)
  position_ids = position_ids.astype(jnp.int32)
  et = edge_types.astype(jnp.int32)

  trows, prows = _embed_gather(word_ids, position_ids, tok_table, pos_table)

  h = None
  for l in range(Wq.shape[0]):
    if l == 0:
      h, q, k, v = _qkv0(trows, prows, Wq[0], Wk[0], Wv[0])
    else:
      q, k, v = _qkv1(h, Wq[l], Wk[l], Wv[l])
    h = _attn_layer(
        q, k, v, h, adj, et, edge_table, Wo[l],
        ln1_g[l].reshape(1, D), ln1_b[l].reshape(1, D),
        W1[l], b1[l].reshape(1, FF), W2[l], b2[l].reshape(1, D),
        ln2_g[l].reshape(1, D), ln2_b[l].reshape(1, D))

  wc_pad = jnp.zeros((D, D), jnp.float32).at[:, :NCLASS].set(Wc)
  bc_pad = jnp.zeros((1, D), jnp.float32).at[:, :NCLASS].set(bc)
  out = _cls_head(h, cls_node.astype(jnp.int32).reshape(B, 1), wc_pad, bc_pad)
  return out[:, :NCLASS]


# speedup vs baseline: 3456.1111x; 1.5439x over previous
import functools

import jax
import jax.numpy as jnp
from jax import lax
from jax.experimental import pallas as pl
from jax.experimental.pallas import tpu as pltpu
from jax.experimental.pallas import tpu_sc as plsc

N = 2048
D = 128
H = 8
DH = 16
FF = 512
NET = 17
NCLASS = 5
B = 16
TN = 512
SCALE = 1.0 / (DH ** 0.5)
QSCALE = SCALE * 1.4426950408889634


def _embed_gather(word_ids, position_ids, tok_table, pos_table):
  info = plsc.get_sparse_core_info()
  nw = info.num_cores * info.num_subcores
  rpw = N // nw

  mesh = plsc.VectorSubcoreMesh(core_axis_name="c", subcore_axis_name="s")

  @functools.partial(
      pl.kernel,
      mesh=mesh,
      out_type=[
          jax.ShapeDtypeStruct((N, D), jnp.float32),
          jax.ShapeDtypeStruct((N, D), jnp.float32),
      ],
      scratch_types=[
          pltpu.VMEM((rpw,), jnp.int32),
          pltpu.VMEM((rpw,), jnp.int32),
          pltpu.VMEM((rpw, D), jnp.float32),
          pltpu.VMEM((rpw, D), jnp.float32),
          pltpu.SemaphoreType.DMA,
          pltpu.SemaphoreType.DMA,
      ],
  )
  def k(wid_hbm, pid_hbm, tok_hbm, pos_hbm, tout_hbm, pout_hbm,
        widx, pidx, trows, prows, sem1, sem2):
    w = lax.axis_index("s") * info.num_cores + lax.axis_index("c")
    base = w * rpw
    pltpu.sync_copy(wid_hbm.at[pl.ds(base, rpw)], widx)
    pltpu.sync_copy(pid_hbm.at[pl.ds(base, rpw)], pidx)
    c1 = pltpu.async_copy(tok_hbm.at[widx], trows, sem1)
    c2 = pltpu.async_copy(pos_hbm.at[pidx], prows, sem2)
    c1.wait()
    c2.wait()
    pltpu.sync_copy(trows, tout_hbm.at[pl.ds(base, rpw)])
    pltpu.sync_copy(prows, pout_hbm.at[pl.ds(base, rpw)])

  return k(word_ids, position_ids, tok_table, pos_table)


VAUG = 2 * D


def _v_augment_t(vv):
  vvt = vv.T
  ones = jnp.ones((1, vvt.shape[1]), jnp.float32)
  zeros = jnp.zeros((DH - 1, vvt.shape[1]), jnp.float32)
  parts = []
  for hh in range(H):
    parts += [vvt[hh * DH:(hh + 1) * DH, :], ones, zeros]
  return jnp.concatenate(parts, axis=0).astype(jnp.bfloat16)


def _qkv_common(hh, wq_ref, wk_ref, wv_ref, q_ref, kt_ref, vt_ref):
  q_ref[...] = (
      jnp.dot(hh, wq_ref[...], preferred_element_type=jnp.float32)
      * QSCALE).astype(jnp.bfloat16)
  kt_ref[...] = jnp.dot(
      hh, wk_ref[...], preferred_element_type=jnp.float32
  ).T.astype(jnp.bfloat16)
  vt_ref[...] = _v_augment_t(
      jnp.dot(hh, wv_ref[...], preferred_element_type=jnp.float32))


def _qkv0_body(t_ref, p_ref, wq_ref, wk_ref, wv_ref,
               h_ref, q_ref, kt_ref, vt_ref):
  hh = t_ref[...] + p_ref[...]
  h_ref[...] = hh
  _qkv_common(hh, wq_ref, wk_ref, wv_ref, q_ref, kt_ref, vt_ref)


def _qkv1_body(h_ref, wq_ref, wk_ref, wv_ref, q_ref, kt_ref, vt_ref):
  _qkv_common(h_ref[...], wq_ref, wk_ref, wv_ref, q_ref, kt_ref, vt_ref)


def _row_spec(dtype=None):
  return pl.BlockSpec((TN, D), lambda i: (i, 0))


def _full_spec(shape):
  return pl.BlockSpec(shape, lambda i: tuple(0 for _ in shape))


_QKV_OUT = [
    jax.ShapeDtypeStruct((N, D), jnp.bfloat16),
    jax.ShapeDtypeStruct((D, N), jnp.bfloat16),
    jax.ShapeDtypeStruct((VAUG, N), jnp.bfloat16),
]

_QKV_OUT_SPECS = [
    pl.BlockSpec((TN, D), lambda i: (i, 0)),
    pl.BlockSpec((D, TN), lambda i: (0, i)),
    pl.BlockSpec((VAUG, TN), lambda i: (0, i)),
]


def _qkv0(t, p, wq, wk, wv):
  return pl.pallas_call(
      _qkv0_body,
      grid=(N // TN,),
      in_specs=[_row_spec(), _row_spec(),
                _full_spec((D, D)), _full_spec((D, D)), _full_spec((D, D))],
      out_specs=[_row_spec()] + _QKV_OUT_SPECS,
      out_shape=[jax.ShapeDtypeStruct((N, D), jnp.float32)] + _QKV_OUT,
  )(t, p, wq, wk, wv)


def _qkv1(h, wq, wk, wv):
  return pl.pallas_call(
      _qkv1_body,
      grid=(N // TN,),
      in_specs=[_row_spec(),
                _full_spec((D, D)), _full_spec((D, D)), _full_spec((D, D))],
      out_specs=list(_QKV_OUT_SPECS),
      out_shape=list(_QKV_OUT),
  )(h, wq, wk, wv)


def _prep_body(adj_ref, et_ref, code_ref):
  code_ref[...] = jnp.where(
      adj_ref[...] > 0.0, et_ref[...], NET).astype(jnp.int8)


def _prep_code(adj, et):
  return pl.pallas_call(
      _prep_body,
      grid=(N // TN,),
      in_specs=[pl.BlockSpec((TN, N), lambda i: (i, 0)),
                pl.BlockSpec((TN, N), lambda i: (i, 0))],
      out_specs=pl.BlockSpec((TN, N), lambda i: (i, 0)),
      out_shape=jax.ShapeDtypeStruct((N, N), jnp.int8),
  )(adj, et)


def _layer_norm(x, g, b, eps=1e-5):
  m = jnp.mean(x, axis=-1, keepdims=True)
  v = jnp.mean((x - m) * (x - m), axis=-1, keepdims=True)
  return (x - m) / jnp.sqrt(v + eps) * g + b


def _attn_body(q_ref, kt_ref, vt_ref, h_ref, code_ref, ek_ref,
               wo_ref, ln1g_ref, ln1b_ref, w1_ref, b1_ref, w2_ref, b2_ref,
               ln2g_ref, ln2b_ref, out_ref):
  qbf = q_ref[...]
  hin = h_ref[...]
  code = code_ref[...].astype(jnp.int32)
  negcol = jnp.full((TN, 1), -1e9, jnp.float32)
  ekbf = ek_ref[...].astype(jnp.bfloat16)

  ctx_parts = []
  den_parts = []
  for hh in range(H):
    sl = slice(hh * DH, (hh + 1) * DH)
    qh = qbf[:, sl]
    kht = kt_ref[sl, :]
    vat = vt_ref[2 * DH * hh:2 * DH * (hh + 1), :]
    s = lax.dot_general(qh, kht, (((1,), (0,)), ((), ())),
                        preferred_element_type=jnp.float32)
    qe = lax.dot_general(qh, ekbf[:, sl], (((1,), (1,)), ((), ())),
                         preferred_element_type=jnp.float32)
    qe = jnp.concatenate([qe, negcol], axis=1)
    bias = jnp.take_along_axis(qe, code, axis=1, mode="promise_in_bounds")
    e = jnp.exp2(s + bias).astype(jnp.bfloat16)
    ca = lax.dot_general(e, vat, (((1,), (1,)), ((), ())),
                         preferred_element_type=jnp.float32)
    ctx_parts.append(ca[:, :DH])
    den_parts.append(ca[:, DH:DH + 1])
  ctx = jnp.concatenate(ctx_parts, axis=1)
  den = jnp.concatenate(den_parts, axis=1)
  sel = (lax.broadcasted_iota(jnp.int32, (H, D), 1) // DH
         == lax.broadcasted_iota(jnp.int32, (H, D), 0)).astype(jnp.float32)
  ctx = ctx * jnp.dot(1.0 / den, sel, preferred_element_type=jnp.float32)

  x = hin + jnp.dot(ctx, wo_ref[...], preferred_element_type=jnp.float32)
  x = _layer_norm(x, ln1g_ref[...], ln1b_ref[...])
  ffn = jnp.dot(
      jnp.maximum(
          jnp.dot(x, w1_ref[...], preferred_element_type=jnp.float32)
          + b1_ref[...],
          0.0),
      w2_ref[...], preferred_element_type=jnp.float32) + b2_ref[...]
  out_ref[...] = _layer_norm(x + ffn, ln2g_ref[...], ln2b_ref[...])


def _attn_layer(q, k, v, h, code, ek, wo, ln1g, ln1b, w1, b1, w2, b2,
                ln2g, ln2b):
  return pl.pallas_call(
      _attn_body,
      grid=(N // TN,),
      in_specs=[
          _row_spec(),
          _full_spec((D, N)),
          _full_spec((VAUG, N)),
          _row_spec(),
          pl.BlockSpec((TN, N), lambda i: (i, 0)),
          _full_spec((NET, D)),
          _full_spec((D, D)),
          _full_spec((1, D)), _full_spec((1, D)),
          _full_spec((D, FF)), _full_spec((1, FF)),
          _full_spec((FF, D)), _full_spec((1, D)),
          _full_spec((1, D)), _full_spec((1, D)),
      ],
      out_specs=_row_spec(),
      out_shape=jax.ShapeDtypeStruct((N, D), jnp.float32),
  )(q, k, v, h, code, ek, wo, ln1g, ln1b, w1, b1, w2, b2, ln2g, ln2b)


def _cls_body(h_ref, cls_ref, wc_ref, bc_ref, out_ref):
  ids = lax.broadcasted_iota(jnp.int32, (B, N), 1)
  oh = (ids == cls_ref[...]).astype(jnp.float32)
  ch = jnp.dot(oh, h_ref[...], preferred_element_type=jnp.float32)
  out_ref[...] = (
      jnp.dot(ch, wc_ref[...], preferred_element_type=jnp.float32)
      + bc_ref[...])


def _cls_head(h, cls_node, wc_pad, bc_pad):
  return pl.pallas_call(
      _cls_body,
      out_shape=jax.ShapeDtypeStruct((B, D), jnp.float32),
  )(h, cls_node, wc_pad, bc_pad)


def kernel(word_ids, position_ids, adj, edge_types, cls_node, tok_table,
           pos_table, edge_table, Wq, Wk, Wv, Wo, ln1_g, ln1_b, W1, b1, W2,
           b2, ln2_g, ln2_b, Wc, bc):
  word_ids = word_ids.astype(jnp.int32)
  position_ids = position_ids.astype(jnp.int32)
  et = edge_types.astype(jnp.int32)

  trows, prows = _embed_gather(word_ids, position_ids, tok_table, pos_table)
  code = _prep_code(adj, et)

  h = None
  for l in range(Wq.shape[0]):
    if l == 0:
      h, q, k, v = _qkv0(trows, prows, Wq[0], Wk[0], Wv[0])
    else:
      q, k, v = _qkv1(h, Wq[l], Wk[l], Wv[l])
    h = _attn_layer(
        q, k, v, h, code, edge_table, Wo[l],
        ln1_g[l].reshape(1, D), ln1_b[l].reshape(1, D),
        W1[l], b1[l].reshape(1, FF), W2[l], b2[l].reshape(1, D),
        ln2_g[l].reshape(1, D), ln2_b[l].reshape(1, D))

  wc_pad = jnp.zeros((D, D), jnp.float32).at[:, :NCLASS].set(Wc)
  bc_pad = jnp.zeros((1, D), jnp.float32).at[:, :NCLASS].set(bc)
  out = _cls_head(h, cls_node.astype(jnp.int32).reshape(B, 1), wc_pad, bc_pad)
  return out[:, :NCLASS]


# speedup vs baseline: 3457.0290x; 1.0003x over previous
import functools

import jax
import jax.numpy as jnp
from jax import lax
from jax.experimental import pallas as pl
from jax.experimental.pallas import tpu as pltpu
from jax.experimental.pallas import tpu_sc as plsc

N = 2048
D = 128
H = 8
DH = 16
FF = 512
NET = 17
NCLASS = 5
B = 16
TN = 512
VAUG = 2 * D
QSCALE = (1.0 / (DH ** 0.5)) * 1.4426950408889634


def _embed_gather(word_ids, position_ids, tok_table, pos_table):
  info = plsc.get_sparse_core_info()
  nw = info.num_cores * info.num_subcores
  rpw = N // nw

  mesh = plsc.VectorSubcoreMesh(core_axis_name="c", subcore_axis_name="s")

  @functools.partial(
      pl.kernel,
      mesh=mesh,
      out_type=[
          jax.ShapeDtypeStruct((N, D), jnp.float32),
          jax.ShapeDtypeStruct((N, D), jnp.float32),
      ],
      scratch_types=[
          pltpu.VMEM((rpw,), jnp.int32),
          pltpu.VMEM((rpw,), jnp.int32),
          pltpu.VMEM((rpw, D), jnp.float32),
          pltpu.VMEM((rpw, D), jnp.float32),
          pltpu.SemaphoreType.DMA,
          pltpu.SemaphoreType.DMA,
      ],
  )
  def k(wid_hbm, pid_hbm, tok_hbm, pos_hbm, tout_hbm, pout_hbm,
        widx, pidx, trows, prows, sem1, sem2):
    w = lax.axis_index("s") * info.num_cores + lax.axis_index("c")
    base = w * rpw
    pltpu.sync_copy(wid_hbm.at[pl.ds(base, rpw)], widx)
    pltpu.sync_copy(pid_hbm.at[pl.ds(base, rpw)], pidx)
    c1 = pltpu.async_copy(tok_hbm.at[widx], trows, sem1)
    c2 = pltpu.async_copy(pos_hbm.at[pidx], prows, sem2)
    c1.wait()
    c2.wait()
    pltpu.sync_copy(trows, tout_hbm.at[pl.ds(base, rpw)])
    pltpu.sync_copy(prows, pout_hbm.at[pl.ds(base, rpw)])

  return k(word_ids, position_ids, tok_table, pos_table)


def _layer_norm(x, g, b, eps=1e-5):
  m = jnp.mean(x, axis=-1, keepdims=True)
  v = jnp.mean((x - m) * (x - m), axis=-1, keepdims=True)
  return (x - m) / jnp.sqrt(v + eps) * g + b


def _v_augment_t(vv):
  vvt = vv.T
  ones = jnp.ones((1, vvt.shape[1]), jnp.float32)
  zeros = jnp.zeros((DH - 1, vvt.shape[1]), jnp.float32)
  parts = []
  for hh in range(H):
    parts += [vvt[hh * DH:(hh + 1) * DH, :], ones, zeros]
  return jnp.concatenate(parts, axis=0).astype(jnp.bfloat16)


def _fill_kv(hf, wk_ref, wv_ref, kt_scr, vat_scr):
  kt_scr[...] = jnp.dot(
      hf, wk_ref[...], preferred_element_type=jnp.float32
  ).T.astype(jnp.bfloat16)
  vat_scr[...] = _v_augment_t(
      jnp.dot(hf, wv_ref[...], preferred_element_type=jnp.float32))


def _attn_ffn(h_tile, code, kt_scr, vat_scr, ek_ref, wq_ref, wo_ref,
              ln1g_ref, ln1b_ref, w1_ref, b1_ref, w2_ref, b2_ref,
              ln2g_ref, ln2b_ref):
  qbf = (jnp.dot(h_tile, wq_ref[...], preferred_element_type=jnp.float32)
         * QSCALE).astype(jnp.bfloat16)
  negcol = jnp.full((TN, 1), -1e9, jnp.float32)
  ekbf = ek_ref[...].astype(jnp.bfloat16)

  ctx_parts = []
  den_parts = []
  for hh in range(H):
    sl = slice(hh * DH, (hh + 1) * DH)
    qh = qbf[:, sl]
    kht = kt_scr[sl, :]
    vat = vat_scr[2 * DH * hh:2 * DH * (hh + 1), :]
    qe = lax.dot_general(qh, ekbf[:, sl], (((1,), (1,)), ((), ())),
                         preferred_element_type=jnp.float32)
    qe = jnp.concatenate([qe, negcol], axis=1)
    bias = jnp.take_along_axis(qe, code, axis=1, mode="promise_in_bounds")
    s = bias + lax.dot_general(qh, kht, (((1,), (0,)), ((), ())),
                               preferred_element_type=jnp.float32)
    e = jnp.exp2(s).astype(jnp.bfloat16)
    ca = lax.dot_general(e, vat, (((1,), (1,)), ((), ())),
                         preferred_element_type=jnp.float32)
    ctx_parts.append(ca[:, :DH])
    den_parts.append(ca[:, DH:DH + 1])
  ctx = jnp.concatenate(ctx_parts, axis=1)
  den = jnp.concatenate(den_parts, axis=1)
  sel = (lax.broadcasted_iota(jnp.int32, (H, D), 1) // DH
         == lax.broadcasted_iota(jnp.int32, (H, D), 0)).astype(jnp.float32)
  ctx = ctx * jnp.dot(1.0 / den, sel, preferred_element_type=jnp.float32)

  x = h_tile + jnp.dot(ctx, wo_ref[...], preferred_element_type=jnp.float32)
  x = _layer_norm(x, ln1g_ref[...], ln1b_ref[...])
  ffn = jnp.dot(
      jnp.maximum(
          jnp.dot(x, w1_ref[...], preferred_element_type=jnp.float32)
          + b1_ref[...],
          0.0),
      w2_ref[...], preferred_element_type=jnp.float32) + b2_ref[...]
  return _layer_norm(x + ffn, ln2g_ref[...], ln2b_ref[...])


def _layer0_body(t_ref, p_ref, adj_ref, et_ref, ek_ref,
                 wq_ref, wk_ref, wv_ref, wo_ref, ln1g_ref, ln1b_ref,
                 w1_ref, b1_ref, w2_ref, b2_ref, ln2g_ref, ln2b_ref,
                 h1_ref, code_ref, hf_scr, kt_scr, vat_scr):
  i = pl.program_id(0)

  @pl.when(i == 0)
  def _():
    hf = t_ref[...] + p_ref[...]
    hf_scr[...] = hf
    _fill_kv(hf, wk_ref, wv_ref, kt_scr, vat_scr)

  code = jnp.where(adj_ref[...] > 0.0, et_ref[...], NET)
  code_ref[...] = code.astype(jnp.int8)
  h_tile = hf_scr[pl.ds(i * TN, TN), :]
  h1_ref[...] = _attn_ffn(
      h_tile, code, kt_scr, vat_scr, ek_ref, wq_ref, wo_ref,
      ln1g_ref, ln1b_ref, w1_ref, b1_ref, w2_ref, b2_ref,
      ln2g_ref, ln2b_ref)


def _layer1_body(h_ref, code_ref, cls_ref, ek_ref,
                 wq_ref, wk_ref, wv_ref, wo_ref, ln1g_ref, ln1b_ref,
                 w1_ref, b1_ref, w2_ref, b2_ref, ln2g_ref, ln2b_ref,
                 wc_ref, bc_ref, logits_ref, kt_scr, vat_scr, cacc_scr):
  i = pl.program_id(0)

  @pl.when(i == 0)
  def _():
    _fill_kv(h_ref[...], wk_ref, wv_ref, kt_scr, vat_scr)

  code = code_ref[...].astype(jnp.int32)
  h_tile = h_ref[pl.ds(i * TN, TN), :]
  x2 = _attn_ffn(
      h_tile, code, kt_scr, vat_scr, ek_ref, wq_ref, wo_ref,
      ln1g_ref, ln1b_ref, w1_ref, b1_ref, w2_ref, b2_ref,
      ln2g_ref, ln2b_ref)

  ids = lax.broadcasted_iota(jnp.int32, (B, TN), 1) + i * TN
  oh = (ids == cls_ref[...]).astype(jnp.float32)
  part = jnp.dot(oh, x2, preferred_element_type=jnp.float32)

  @pl.when(i == 0)
  def _():
    cacc_scr[...] = part

  @pl.when(i > 0)
  def _():
    cacc_scr[...] = cacc_scr[...] + part

  @pl.when(i == pl.num_programs(0) - 1)
  def _():
    logits_ref[...] = (
        jnp.dot(cacc_scr[...], wc_ref[...],
                preferred_element_type=jnp.float32) + bc_ref[...])


def _row_tile():
  return pl.BlockSpec((TN, N), lambda i: (i, 0))


def _full_spec(shape):
  return pl.BlockSpec(shape, lambda i: tuple(0 for _ in shape))


def _weight_specs():
  return [
      _full_spec((NET, D)),
      _full_spec((D, D)), _full_spec((D, D)),
      _full_spec((D, D)), _full_spec((D, D)),
      _full_spec((1, D)), _full_spec((1, D)),
      _full_spec((D, FF)), _full_spec((1, FF)),
      _full_spec((FF, D)), _full_spec((1, D)),
      _full_spec((1, D)), _full_spec((1, D)),
  ]


_KV_SCRATCH = [
    pltpu.VMEM((D, N), jnp.bfloat16),
    pltpu.VMEM((VAUG, N), jnp.bfloat16),
]


def _layer0(t, p, adj, et, ek, *w):
  return pl.pallas_call(
      _layer0_body,
      grid=(N // TN,),
      in_specs=[_full_spec((N, D)), _full_spec((N, D)),
                _row_tile(), _row_tile()] + _weight_specs(),
      out_specs=[pl.BlockSpec((TN, D), lambda i: (i, 0)), _row_tile()],
      out_shape=[jax.ShapeDtypeStruct((N, D), jnp.float32),
                 jax.ShapeDtypeStruct((N, N), jnp.int8)],
      scratch_shapes=[pltpu.VMEM((N, D), jnp.float32)] + _KV_SCRATCH,
  )(t, p, adj, et, ek, *w)


def _layer1(h, code, cls_node, ek, *w):
  return pl.pallas_call(
      _layer1_body,
      grid=(N // TN,),
      in_specs=[_full_spec((N, D)), _row_tile(), _full_spec((B, 1))]
      + _weight_specs() + [_full_spec((D, D)), _full_spec((1, D))],
      out_specs=_full_spec((B, D)),
      out_shape=jax.ShapeDtypeStruct((B, D), jnp.float32),
      scratch_shapes=_KV_SCRATCH + [pltpu.VMEM((B, D), jnp.float32)],
  )(h, code, cls_node, ek, *w)


def kernel(word_ids, position_ids, adj, edge_types, cls_node, tok_table,
           pos_table, edge_table, Wq, Wk, Wv, Wo, ln1_g, ln1_b, W1, b1, W2,
           b2, ln2_g, ln2_b, Wc, bc):
  word_ids = word_ids.astype(jnp.int32)
  position_ids = position_ids.astype(jnp.int32)
  et = edge_types.astype(jnp.int32)

  trows, prows = _embed_gather(word_ids, position_ids, tok_table, pos_table)

  def lw(l):
    return (edge_table, Wq[l], Wk[l], Wv[l], Wo[l],
            ln1_g[l].reshape(1, D), ln1_b[l].reshape(1, D),
            W1[l], b1[l].reshape(1, FF), W2[l], b2[l].reshape(1, D),
            ln2_g[l].reshape(1, D), ln2_b[l].reshape(1, D))

  h1, code = _layer0(trows, prows, adj, et, *lw(0))

  wc_pad = jnp.zeros((D, D), jnp.float32).at[:, :NCLASS].set(Wc)
  bc_pad = jnp.zeros((1, D), jnp.float32).at[:, :NCLASS].set(bc)
  logits = _layer1(h1, code, cls_node.astype(jnp.int32).reshape(B, 1),
                   *lw(1), wc_pad, bc_pad)
  return logits[:, :NCLASS]


# speedup vs baseline: 3495.9595x; 1.0113x over previous
import functools

import jax
import jax.numpy as jnp
from jax import lax
from jax.experimental import pallas as pl
from jax.experimental.pallas import tpu as pltpu
from jax.experimental.pallas import tpu_sc as plsc

N = 2048
D = 128
H = 8
DH = 16
FF = 512
NET = 17
NCLASS = 5
B = 16
TN = 512
VAUG = 2 * D
QSCALE = (1.0 / (DH ** 0.5)) * 1.4426950408889634


def _embed_gather(word_ids, position_ids, tok_table, pos_table):
  info = plsc.get_sparse_core_info()
  nw = info.num_cores * info.num_subcores
  rpw = N // nw

  mesh = plsc.VectorSubcoreMesh(core_axis_name="c", subcore_axis_name="s")

  @functools.partial(
      pl.kernel,
      mesh=mesh,
      out_type=[
          jax.ShapeDtypeStruct((N, D), jnp.float32),
          jax.ShapeDtypeStruct((N, D), jnp.float32),
      ],
      scratch_types=[
          pltpu.VMEM((rpw,), jnp.int32),
          pltpu.VMEM((rpw,), jnp.int32),
          pltpu.VMEM((rpw, D), jnp.float32),
          pltpu.VMEM((rpw, D), jnp.float32),
          pltpu.SemaphoreType.DMA,
          pltpu.SemaphoreType.DMA,
      ],
  )
  def k(wid_hbm, pid_hbm, tok_hbm, pos_hbm, tout_hbm, pout_hbm,
        widx, pidx, trows, prows, sem1, sem2):
    w = lax.axis_index("s") * info.num_cores + lax.axis_index("c")
    base = w * rpw
    pltpu.sync_copy(wid_hbm.at[pl.ds(base, rpw)], widx)
    pltpu.sync_copy(pid_hbm.at[pl.ds(base, rpw)], pidx)
    c1 = pltpu.async_copy(tok_hbm.at[widx], trows, sem1)
    c2 = pltpu.async_copy(pos_hbm.at[pidx], prows, sem2)
    c1.wait()
    c2.wait()
    pltpu.sync_copy(trows, tout_hbm.at[pl.ds(base, rpw)])
    pltpu.sync_copy(prows, pout_hbm.at[pl.ds(base, rpw)])

  return k(word_ids, position_ids, tok_table, pos_table)


def _layer_norm(x, g, b, eps=1e-5):
  m = jnp.mean(x, axis=-1, keepdims=True)
  v = jnp.mean((x - m) * (x - m), axis=-1, keepdims=True)
  return (x - m) / jnp.sqrt(v + eps) * g + b


def _fill_kv(hf, wk_ref, wv_ref, kt_scr, vat_scr):
  kt_scr[...] = lax.dot_general(
      wk_ref[...], hf, (((0,), (1,)), ((), ())),
      preferred_element_type=jnp.float32).astype(jnp.bfloat16)
  vvt = lax.dot_general(
      wv_ref[...], hf, (((0,), (1,)), ((), ())),
      preferred_element_type=jnp.float32)
  ones = jnp.ones((1, N), jnp.float32)
  zeros = jnp.zeros((DH - 1, N), jnp.float32)
  parts = []
  for hh in range(H):
    parts += [vvt[hh * DH:(hh + 1) * DH, :], ones, zeros]
  vat_scr[...] = jnp.concatenate(parts, axis=0).astype(jnp.bfloat16)


def _attn_ffn(h_tile, code, kt_scr, vat_scr, ek_ref, wq_ref, wo_ref,
              ln1g_ref, ln1b_ref, w1_ref, b1_ref, w2_ref, b2_ref,
              ln2g_ref, ln2b_ref):
  qbf = (jnp.dot(h_tile, wq_ref[...], preferred_element_type=jnp.float32)
         * QSCALE).astype(jnp.bfloat16)
  negcol = jnp.full((TN, 1), -1e9, jnp.float32)
  ekbf = ek_ref[...].astype(jnp.bfloat16)

  ctx_parts = []
  den_parts = []
  for hh in range(H):
    sl = slice(hh * DH, (hh + 1) * DH)
    qh = qbf[:, sl]
    kht = kt_scr[sl, :]
    vat = vat_scr[2 * DH * hh:2 * DH * (hh + 1), :]
    qe = lax.dot_general(qh, ekbf[:, sl], (((1,), (1,)), ((), ())),
                         preferred_element_type=jnp.float32)
    qe = jnp.concatenate([qe, negcol], axis=1)
    bias = jnp.take_along_axis(qe, code, axis=1, mode="promise_in_bounds")
    s = bias + lax.dot_general(qh, kht, (((1,), (0,)), ((), ())),
                               preferred_element_type=jnp.float32)
    e = jnp.exp2(s).astype(jnp.bfloat16)
    ca = lax.dot_general(e, vat, (((1,), (1,)), ((), ())),
                         preferred_element_type=jnp.float32)
    ctx_parts.append(ca)
  ca_all = jnp.concatenate(ctx_parts, axis=1)
  j32 = lax.broadcasted_iota(jnp.int32, (VAUG, H), 0)
  hidx = lax.broadcasted_iota(jnp.int32, (VAUG, H), 1)
  pick = (j32 == hidx * 2 * DH + DH).astype(jnp.float32)
  den = jnp.dot(ca_all, pick, preferred_element_type=jnp.float32)
  sel = (lax.broadcasted_iota(jnp.int32, (H, VAUG), 1) // (2 * DH)
         == lax.broadcasted_iota(jnp.int32, (H, VAUG), 0)).astype(jnp.float32)
  ctx = ca_all * jnp.dot(1.0 / den, sel, preferred_element_type=jnp.float32)

  x = h_tile + jnp.dot(ctx, wo_ref[...], preferred_element_type=jnp.float32)
  x = _layer_norm(x, ln1g_ref[...], ln1b_ref[...])
  ffn = jnp.dot(
      jnp.maximum(
          jnp.dot(x, w1_ref[...], preferred_element_type=jnp.float32)
          + b1_ref[...],
          0.0),
      w2_ref[...], preferred_element_type=jnp.float32) + b2_ref[...]
  return _layer_norm(x + ffn, ln2g_ref[...], ln2b_ref[...])


def _layer0_body(t_ref, p_ref, adj_ref, et_ref, ek_ref,
                 wq_ref, wk_ref, wv_ref, wo_ref, ln1g_ref, ln1b_ref,
                 w1_ref, b1_ref, w2_ref, b2_ref, ln2g_ref, ln2b_ref,
                 h1_ref, code_ref, hf_scr, kt_scr, vat_scr):
  i = pl.program_id(0)

  @pl.when(i == 0)
  def _():
    hf = t_ref[...] + p_ref[...]
    hf_scr[...] = hf
    _fill_kv(hf, wk_ref, wv_ref, kt_scr, vat_scr)

  code = jnp.where(adj_ref[...] > 0.0, et_ref[...], NET)
  code_ref[...] = code.astype(jnp.int8)
  h_tile = hf_scr[pl.ds(i * TN, TN), :]
  h1_ref[...] = _attn_ffn(
      h_tile, code, kt_scr, vat_scr, ek_ref, wq_ref, wo_ref,
      ln1g_ref, ln1b_ref, w1_ref, b1_ref, w2_ref, b2_ref,
      ln2g_ref, ln2b_ref)


def _layer1_body(h_ref, code_ref, cls_ref, ek_ref,
                 wq_ref, wk_ref, wv_ref, wo_ref, ln1g_ref, ln1b_ref,
                 w1_ref, b1_ref, w2_ref, b2_ref, ln2g_ref, ln2b_ref,
                 wc_ref, bc_ref, logits_ref, kt_scr, vat_scr, cacc_scr):
  i = pl.program_id(0)

  @pl.when(i == 0)
  def _():
    _fill_kv(h_ref[...], wk_ref, wv_ref, kt_scr, vat_scr)

  code = code_ref[...].astype(jnp.int32)
  h_tile = h_ref[pl.ds(i * TN, TN), :]
  x2 = _attn_ffn(
      h_tile, code, kt_scr, vat_scr, ek_ref, wq_ref, wo_ref,
      ln1g_ref, ln1b_ref, w1_ref, b1_ref, w2_ref, b2_ref,
      ln2g_ref, ln2b_ref)

  ids = lax.broadcasted_iota(jnp.int32, (B, TN), 1) + i * TN
  oh = (ids == cls_ref[...]).astype(jnp.float32)
  part = jnp.dot(oh, x2, preferred_element_type=jnp.float32)

  @pl.when(i == 0)
  def _():
    cacc_scr[...] = part

  @pl.when(i > 0)
  def _():
    cacc_scr[...] = cacc_scr[...] + part

  @pl.when(i == pl.num_programs(0) - 1)
  def _():
    logits_ref[...] = (
        jnp.dot(cacc_scr[...], wc_ref[...],
                preferred_element_type=jnp.float32) + bc_ref[...])


def _row_tile():
  return pl.BlockSpec((TN, N), lambda i: (i, 0))


def _full_spec(shape):
  return pl.BlockSpec(shape, lambda i: tuple(0 for _ in shape))


def _weight_specs():
  return [
      _full_spec((NET, D)),
      _full_spec((D, D)), _full_spec((D, D)),
      _full_spec((D, D)), _full_spec((VAUG, D)),
      _full_spec((1, D)), _full_spec((1, D)),
      _full_spec((D, FF)), _full_spec((1, FF)),
      _full_spec((FF, D)), _full_spec((1, D)),
      _full_spec((1, D)), _full_spec((1, D)),
  ]


_KV_SCRATCH = [
    pltpu.VMEM((D, N), jnp.bfloat16),
    pltpu.VMEM((VAUG, N), jnp.bfloat16),
]


def _layer0(t, p, adj, et, ek, *w):
  return pl.pallas_call(
      _layer0_body,
      grid=(N // TN,),
      in_specs=[_full_spec((N, D)), _full_spec((N, D)),
                _row_tile(), _row_tile()] + _weight_specs(),
      out_specs=[pl.BlockSpec((TN, D), lambda i: (i, 0)), _row_tile()],
      out_shape=[jax.ShapeDtypeStruct((N, D), jnp.float32),
                 jax.ShapeDtypeStruct((N, N), jnp.int8)],
      scratch_shapes=[pltpu.VMEM((N, D), jnp.float32)] + _KV_SCRATCH,
  )(t, p, adj, et, ek, *w)


def _layer1(h, code, cls_node, ek, *w):
  return pl.pallas_call(
      _layer1_body,
      grid=(N // TN,),
      in_specs=[_full_spec((N, D)), _row_tile(), _full_spec((B, 1))]
      + _weight_specs() + [_full_spec((D, D)), _full_spec((1, D))],
      out_specs=_full_spec((B, D)),
      out_shape=jax.ShapeDtypeStruct((B, D), jnp.float32),
      scratch_shapes=_KV_SCRATCH + [pltpu.VMEM((B, D), jnp.float32)],
  )(h, code, cls_node, ek, *w)


def kernel(word_ids, position_ids, adj, edge_types, cls_node, tok_table,
           pos_table, edge_table, Wq, Wk, Wv, Wo, ln1_g, ln1_b, W1, b1, W2,
           b2, ln2_g, ln2_b, Wc, bc):
  word_ids = word_ids.astype(jnp.int32)
  position_ids = position_ids.astype(jnp.int32)
  et = edge_types.astype(jnp.int32)

  trows, prows = _embed_gather(word_ids, position_ids, tok_table, pos_table)

  def lw(l):
    woa = jnp.concatenate(
        [Wo[l].reshape(H, DH, D), jnp.zeros((H, DH, D), jnp.float32)],
        axis=1).reshape(VAUG, D)
    return (edge_table, Wq[l], Wk[l], Wv[l], woa,
            ln1_g[l].reshape(1, D), ln1_b[l].reshape(1, D),
            W1[l], b1[l].reshape(1, FF), W2[l], b2[l].reshape(1, D),
            ln2_g[l].reshape(1, D), ln2_b[l].reshape(1, D))

  h1, code = _layer0(trows, prows, adj, et, *lw(0))

  wc_pad = jnp.zeros((D, D), jnp.float32).at[:, :NCLASS].set(Wc)
  bc_pad = jnp.zeros((1, D), jnp.float32).at[:, :NCLASS].set(bc)
  logits = _layer1(h1, code, cls_node.astype(jnp.int32).reshape(B, 1),
                   *lw(1), wc_pad, bc_pad)
  return logits[:, :NCLASS]
